# Initial kernel scaffold; baseline (speedup 1.0000x reference)
import jax
import jax.numpy as jnp
from jax.experimental import pallas as pl


def kernel(x, blosum):
    raise NotImplementedError("write your pallas kernel here")


# speedup vs baseline: 6.2010x; 6.2010x over previous
import functools

import jax
import jax.numpy as jnp
from jax import lax
from jax.experimental import pallas as pl
from jax.experimental.pallas import tpu as pltpu
from jax.experimental.pallas import tpu_sc as plsc

NUM_CORES = 2
NUM_SUBCORES = 16
NUM_WORKERS = NUM_CORES * NUM_SUBCORES

IDX_COLS = 128
IDX_ROWS = 16
CHUNK = IDX_ROWS * IDX_COLS


def _emb_kernel(n_chunks, v, d, table_hbm, idx_hbm, out_hbm,
                table_sh, idx_v, rows_v, sem):
    cid = lax.axis_index("c")
    sid = lax.axis_index("s")
    wid = sid * NUM_CORES + cid

    @pl.when(sid == 0)
    def _():
        pltpu.sync_copy(table_hbm, table_sh)

    plsc.subcore_barrier()

    def chunk_body(c, carry):
        base = (wid * n_chunks + c) * CHUNK
        row0 = pl.multiple_of(base // IDX_COLS, IDX_ROWS)
        pltpu.sync_copy(idx_hbm.at[pl.ds(row0, IDX_ROWS)], idx_v)
        copies = [
            pltpu.async_copy(
                table_sh.at[idx_v.at[j]],
                rows_v.at[pl.ds(j * IDX_COLS, IDX_COLS)],
                sem,
            )
            for j in range(IDX_ROWS)
        ]
        for cp in copies:
            cp.wait()
        pltpu.sync_copy(rows_v, out_hbm.at[pl.ds(base, CHUNK)])
        return carry

    lax.fori_loop(0, n_chunks, chunk_body, 0)


def kernel(x, blosum):
    b0, s = x.shape
    v, d = blosum.shape
    b = b0 * s
    assert b % (NUM_WORKERS * CHUNK) == 0
    n_chunks = b // (NUM_WORKERS * CHUNK)

    idx = x.reshape(b // IDX_COLS, IDX_COLS).astype(jnp.int32)

    mesh = plsc.VectorSubcoreMesh(
        core_axis_name="c", subcore_axis_name="s",
        num_cores=NUM_CORES, num_subcores=NUM_SUBCORES,
    )
    emb = pl.kernel(
        functools.partial(_emb_kernel, n_chunks, v, d),
        out_type=jax.ShapeDtypeStruct((b, d), jnp.float32),
        mesh=mesh,
        scratch_types=[
            pltpu.VMEM_SHARED((v, d), jnp.float32),
            pltpu.VMEM((IDX_ROWS, IDX_COLS), jnp.int32),
            pltpu.VMEM((CHUNK, d), jnp.float32),
            pltpu.SemaphoreType.DMA,
        ],
        compiler_params=pltpu.CompilerParams(use_tc_tiling_on_sc=False),
    )
    out = emb(blosum, idx)
    return out.reshape(b0, s, d)
